# baseline (speedup 1.0000x reference)
import functools

import jax
import jax.numpy as jnp
import numpy as np
from jax import lax
from jax.experimental import pallas as pl
from jax.experimental.pallas import tpu as pltpu
from jax.experimental.pallas import tpu_sc as plsc

NC = 2
NS = 16
NW = NC * NS

STREAM = 128
NBUF = 4


def _positional_encoding(model_size, sequence_length):
    pos = np.arange(sequence_length, dtype=np.float64)[:, None]
    i = np.arange(model_size, dtype=np.float64)[None, :]
    exponent = np.where(i % 2 == 0, i, i - 1) / model_size
    angle = pos / np.power(10000.0, exponent)
    pe = np.where(i % 2 == 0, np.sin(angle), np.cos(angle))
    return pe.astype(np.float32)


def _pe_phases(dim, seq, dim_pad):
    n_phases = seq // np.gcd(STREAM, seq)
    pe = _positional_encoding(dim, seq)
    out = np.zeros((n_phases, STREAM, dim_pad), dtype=np.float32)
    for k in range(n_phases):
        o = (STREAM * k) % seq
        rows = (o + np.arange(STREAM)) % seq
        out[k, :, :dim] = pe[rows]
    return out


@functools.partial(jax.jit, static_argnums=(3,))
def _sc_embed(idx, pe_all, table, n_streams):
    n_phases, _, dpad = pe_all.shape
    b_per_w = n_streams * STREAM
    B = NW * b_per_w

    mesh = plsc.VectorSubcoreMesh(
        core_axis_name="c", subcore_axis_name="s",
        num_cores=NC, num_subcores=NS)

    @functools.partial(
        pl.kernel,
        mesh=mesh,
        out_type=jax.ShapeDtypeStruct((B, dpad), jnp.float32),
        scratch_types=[
            pltpu.VMEM((n_streams, STREAM), jnp.int32),
            pltpu.VMEM((NBUF, STREAM, dpad), jnp.float32),
            pltpu.VMEM_SHARED((n_phases, STREAM, dpad), jnp.float32),
            pltpu.SemaphoreType.DMA((NBUF,)),
            pltpu.SemaphoreType.DMA((NBUF,)),
            pltpu.SemaphoreType.DMA((NBUF,)),
        ],
    )
    def k(idx_hbm, pe_hbm, table_hbm, out_hbm,
          idx_v, rows_v, pe_sh, psem, gsem, osem):
        tview = table_hbm
        sid = lax.axis_index("s")
        wid = sid * NC + lax.axis_index("c")
        base = wid * b_per_w

        @pl.when(sid == 0)
        def _():
            pltpu.sync_copy(pe_hbm, pe_sh)
        pltpu.sync_copy(idx_hbm.at[wid], idx_v)
        plsc.subcore_barrier()

        def mi_body(mi, _):
            for b in range(NBUF):
                j = mi * NBUF + b

                @pl.when(mi >= 1)
                def _(b=b):
                    pltpu.make_async_copy(
                        rows_v.at[b], out_hbm.at[pl.ds(0, STREAM)],
                        osem.at[b]).wait()
                pltpu.async_copy(
                    pe_sh.at[lax.rem(j, n_phases)], rows_v.at[b], psem.at[b])

            for b in range(NBUF):
                j = mi * NBUF + b
                pltpu.make_async_copy(
                    pe_sh.at[0], rows_v.at[b], psem.at[b]).wait()
                pltpu.async_copy(
                    tview.at[idx_v.at[j]], rows_v.at[b],
                    gsem.at[b], add=True)

            for b in range(NBUF):
                j = mi * NBUF + b
                pltpu.make_async_copy(
                    tview.at[idx_v.at[j]], rows_v.at[b], gsem.at[b]).wait()
                pltpu.async_copy(
                    rows_v.at[b],
                    out_hbm.at[pl.ds(base + j * STREAM, STREAM)], osem.at[b])
            return ()

        lax.fori_loop(0, n_streams // NBUF, mi_body, ())

        for b in range(NBUF):
            pltpu.make_async_copy(
                rows_v.at[b], out_hbm.at[pl.ds(0, STREAM)], osem.at[b]).wait()

    return k(idx, pe_all, table)


def kernel(inputs, table):
    batch, seq = inputs.shape
    vocab, dim = table.shape
    B = batch * seq
    dpad = 128
    assert B % (NW * STREAM) == 0
    n_streams = B // (NW * STREAM)
    assert (n_streams * STREAM) % seq == 0

    table128 = jnp.pad(table, ((0, 0), (0, dpad - dim)))
    idx = inputs.astype(jnp.int32).reshape(NW, n_streams, STREAM)
    pe_all = jnp.asarray(_pe_phases(dim, seq, dpad))
    out = _sc_embed(idx, pe_all, table128, n_streams)
    return out.reshape(batch, seq, dpad)[:, :, :dim]
